# baseline (speedup 1.0000x reference)
import functools

import jax
import jax.numpy as jnp
from jax import lax
from jax.experimental import pallas as pl
from jax.experimental.pallas import tpu as pltpu
from jax.experimental.pallas import tpu_sc as plsc

_B = 16
_MAX_LEN = 4096
_D = 1024

_CH = 512

_HALF = _D // 2
_NV = _HALF // 16
_R = 32
_RSH = 5
_NBUF = 4
_NRANGE = 16
_NSLAB = _HALF // 128


def _zero_vec():
    return jnp.zeros((16,), jnp.float32)


def _tc_sums(x, lengths):
    def body(len_ref, x_ref, o_ref, vb0, vb1, sem0, sem1):
        for b in range(_B):
            ln = len_ref[b]
            nf = ln >> 9

            o_ref[pl.ds(b, 1), :] = jnp.zeros((1, _D), jnp.float32)

            @pl.when(nf > 0)
            def _go():
                npair = lax.shift_right_arithmetic(nf + 1, 1)

                def src(cc):
                    t0 = pl.multiple_of(
                        _CH * jnp.clip(cc, 0, nf - 1), _CH)
                    return x_ref.at[b, pl.ds(t0, _CH), :]

                pltpu.make_async_copy(src(jnp.int32(0)), vb0, sem0).start()
                pltpu.make_async_copy(src(jnp.int32(1)), vb1, sem1).start()

                def pairs(p, acc):
                    c0 = 2 * p
                    pltpu.make_async_copy(src(c0), vb0, sem0).wait()
                    acc = acc + jnp.sum(vb0[...], axis=0, keepdims=True)

                    @pl.when(p + 1 < npair)
                    def _i0():
                        pltpu.make_async_copy(src(c0 + 2), vb0, sem0).start()

                    pltpu.make_async_copy(src(c0 + 1), vb1, sem1).wait()
                    s1 = jnp.sum(vb1[...], axis=0, keepdims=True)
                    acc = acc + jnp.where(c0 + 1 < nf, s1, 0.0)

                    @pl.when(p + 1 < npair)
                    def _i1():
                        pltpu.make_async_copy(src(c0 + 3), vb1, sem1).start()

                    return acc

                acc = lax.fori_loop(0, npair, pairs,
                                    jnp.zeros((1, _D), jnp.float32))
                o_ref[pl.ds(b, 1), :] = acc

    return pl.pallas_call(
        body,
        in_specs=[pl.BlockSpec(memory_space=pltpu.SMEM),
                  pl.BlockSpec(memory_space=pl.ANY)],
        out_specs=pl.BlockSpec(memory_space=pltpu.VMEM),
        out_shape=jax.ShapeDtypeStruct((_B, _D), jnp.float32),
        scratch_shapes=[
            pltpu.VMEM((_CH, _D), jnp.float32),
            pltpu.VMEM((_CH, _D), jnp.float32),
            pltpu.SemaphoreType.DMA,
            pltpu.SemaphoreType.DMA,
        ],
    )(lengths, x)


def _sc_sums(x, lengths):
    mesh = plsc.VectorSubcoreMesh(core_axis_name="c", subcore_axis_name="s")

    @functools.partial(
        pl.kernel,
        out_type=jax.ShapeDtypeStruct((_B, _D), jnp.float32),
        mesh=mesh,
        scratch_types=(
            [pltpu.VMEM((32,), jnp.int32),
             pltpu.SMEM((16,), jnp.int32)]
            + [pltpu.VMEM((_R, _HALF), jnp.float32) for _ in range(_NBUF)]
            + [pltpu.VMEM((_B, _HALF), jnp.float32),
               pltpu.VMEM_SHARED((_NRANGE, _B, _HALF), jnp.float32),
               pltpu.VMEM((_NRANGE, 8, 128), jnp.float32),
               pltpu.VMEM((8, 128), jnp.float32)]
            + [pltpu.SemaphoreType.DMA for _ in range(_NBUF)]
        ),
    )
    def run(x_hbm, len_hbm, out_hbm, len_v, starts_s, *rest):
        bufs = rest[:_NBUF]
        stage, shared, bufb, outb = rest[_NBUF:_NBUF + 4]
        sems = rest[_NBUF + 4:]
        c = lax.axis_index("c")
        s = lax.axis_index("s")
        col0 = c * _HALF

        pltpu.sync_copy(len_hbm, len_v.at[pl.ds(0, 16)])

        total = jnp.int32(0)
        for b in range(_B):
            starts_s[b] = total
            lb = len_v[pl.ds(b, 16)][0]
            total = total + (lb - (lb & (-_CH)))

        lo = lax.shift_right_arithmetic(s * total, 4)
        hi = lax.shift_right_arithmetic((s + 1) * total, 4)

        zero = _zero_vec()

        def seq_body(b, carry):
            start = starts_s[b]
            lb = len_v[pl.ds(b, 16)][0]
            st = lb & (-_CH)
            t_lo = st + jnp.clip(lo - start, 0, lb - st)
            t_hi = st + jnp.clip(hi - start, 0, lb - st)
            nrows = t_hi - t_lo

            for j in range(_NV):
                stage[b, pl.ds(16 * j, 16)] = zero

            @pl.when(nrows > 0)
            def _process():
                a_lo = t_lo & (-8)
                nch = lax.shift_right_arithmetic(
                    t_hi - a_lo + (_R - 1), _RSH)
                ngrp = lax.shift_right_arithmetic(nch + (_NBUF - 1), 2)

                def src(g):
                    t0 = pl.multiple_of(
                        jnp.minimum(a_lo + g * _R, _MAX_LEN - _R), 8)
                    return x_hbm.at[b, pl.ds(t0, _R), pl.ds(col0, _HALF)]

                for i in range(_NBUF):
                    pltpu.async_copy(src(i), bufs[i], sems[i])

                def accum(buf, g, acc):
                    gstart = a_lo + g * _R
                    t0 = jnp.minimum(gstart, _MAX_LEN - _R)
                    k_lo = jnp.maximum(t_lo, gstart) - t0
                    k_hi = jnp.minimum(t_hi, gstart + _R) - t0

                    def row(k, a):
                        return tuple(a[j] + buf[k, pl.ds(16 * j, 16)]
                                     for j in range(_NV))

                    return lax.fori_loop(k_lo, k_hi, row, acc)

                def grp(p, acc):
                    g0 = _NBUF * p
                    for i in range(_NBUF):
                        pltpu.make_async_copy(src(g0 + i), bufs[i],
                                              sems[i]).wait()
                        acc = accum(bufs[i], g0 + i, acc)

                        @pl.when(p + 1 < ngrp)
                        def _issue():
                            pltpu.async_copy(src(g0 + _NBUF + i), bufs[i],
                                             sems[i])
                    return acc

                acc = lax.fori_loop(0, ngrp, grp,
                                    tuple(zero for _ in range(_NV)))
                for j in range(_NV):
                    stage[b, pl.ds(16 * j, 16)] = acc[j]

            return carry

        lax.fori_loop(0, _B, seq_body, jnp.int32(0))

        pltpu.sync_copy(stage, shared.at[s])
        plsc.subcore_barrier()

        @pl.when(s < 2 * _NSLAB)
        def _combine():
            g = s // _NSLAB
            e = s % _NSLAB
            row0 = 8 * g
            cb = 128 * e

            pltpu.sync_copy(
                shared.at[pl.ds(0, _NRANGE), pl.ds(row0, 8), pl.ds(cb, 128)],
                bufb)

            for q in range(8):
                for j in range(8):
                    acc = _zero_vec()
                    for k in range(_NRANGE):
                        acc = acc + bufb[k, q, pl.ds(16 * j, 16)]
                    outb[q, pl.ds(16 * j, 16)] = acc
            pltpu.sync_copy(
                outb,
                out_hbm.at[pl.ds(row0, 8), pl.ds(col0 + cb, 128)])

    return run(x, lengths)


def kernel(input_sequences, sequence_lengths):
    lengths = sequence_lengths.astype(jnp.int32)
    sc = _sc_sums(input_sequences, lengths)
    tc = _tc_sums(input_sequences, lengths)
    return (tc + sc) / lengths.astype(jnp.float32)[:, None]


# speedup vs baseline: 1.3728x; 1.3728x over previous
import functools

import jax
import jax.numpy as jnp
from jax import lax
from jax.experimental import pallas as pl
from jax.experimental.pallas import tpu as pltpu
from jax.experimental.pallas import tpu_sc as plsc

_B = 16
_MAX_LEN = 4096
_D = 1024

_CH = 512
_TBUF = 3

_HALF = _D // 2
_NV = _HALF // 16
_R = 32
_RSH = 5
_NBUF = 4
_NRANGE = 16
_NSLAB = _HALF // 128


def _zero_vec():
    return jnp.zeros((16,), jnp.float32)


def _tc_sums(x, lengths):
    def body(len_ref, x_ref, o_ref, *rest):
        vbs = rest[:_TBUF]
        sems = rest[_TBUF:]

        pref = []
        tot = jnp.int32(0)
        for b in range(_B):
            pref.append(tot)
            tot = tot + (len_ref[b] >> 9)
        n_chunks = tot

        for b in range(_B):
            o_ref[pl.ds(b, 1), :] = jnp.zeros((1, _D), jnp.float32)

        def locate(c):
            bsel = jnp.int32(0)
            fsel = pref[0]
            for bb in range(1, _B):
                take = c >= pref[bb]
                bsel = jnp.where(take, jnp.int32(bb), bsel)
                fsel = jnp.where(take, pref[bb], fsel)
            return bsel, fsel

        def src(c):
            cc = jnp.clip(c, 0, jnp.maximum(n_chunks - 1, 0))
            bsel, fsel = locate(cc)
            t0 = pl.multiple_of(_CH * (cc - fsel), _CH)
            return x_ref.at[bsel, pl.ds(t0, _CH), :]

        for i in range(_TBUF):
            pltpu.make_async_copy(src(jnp.int32(i)), vbs[i], sems[i]).start()

        ngrp = lax.div(n_chunks + (_TBUF - 1), jnp.int32(_TBUF))

        def grp(p, carry):
            g0 = _TBUF * p
            for i in range(_TBUF):
                c = g0 + i
                pltpu.make_async_copy(src(c), vbs[i], sems[i]).wait()
                s = jnp.sum(vbs[i][...], axis=0, keepdims=True)
                s = jnp.where(c < n_chunks, s, 0.0)
                bsel, _ = locate(jnp.clip(c, 0, jnp.maximum(n_chunks - 1, 0)))
                o_ref[pl.ds(bsel, 1), :] += s

                @pl.when(p + 1 < ngrp)
                def _issue():
                    pltpu.make_async_copy(src(c + _TBUF), vbs[i],
                                          sems[i]).start()
            return carry

        lax.fori_loop(0, ngrp, grp, jnp.int32(0))

    return pl.pallas_call(
        body,
        in_specs=[pl.BlockSpec(memory_space=pltpu.SMEM),
                  pl.BlockSpec(memory_space=pl.ANY)],
        out_specs=pl.BlockSpec(memory_space=pltpu.VMEM),
        out_shape=jax.ShapeDtypeStruct((_B, _D), jnp.float32),
        scratch_shapes=(
            [pltpu.VMEM((_CH, _D), jnp.float32) for _ in range(_TBUF)]
            + [pltpu.SemaphoreType.DMA for _ in range(_TBUF)]
        ),
    )(lengths, x)


def _sc_sums(x, lengths):
    mesh = plsc.VectorSubcoreMesh(core_axis_name="c", subcore_axis_name="s")

    @functools.partial(
        pl.kernel,
        out_type=jax.ShapeDtypeStruct((_B, _D), jnp.float32),
        mesh=mesh,
        scratch_types=(
            [pltpu.VMEM((32,), jnp.int32),
             pltpu.SMEM((16,), jnp.int32)]
            + [pltpu.VMEM((_R, _HALF), jnp.float32) for _ in range(_NBUF)]
            + [pltpu.VMEM((_B, _HALF), jnp.float32),
               pltpu.VMEM_SHARED((_NRANGE, _B, _HALF), jnp.float32),
               pltpu.VMEM((_NRANGE, 8, 128), jnp.float32),
               pltpu.VMEM((8, 128), jnp.float32)]
            + [pltpu.SemaphoreType.DMA for _ in range(_NBUF)]
        ),
    )
    def run(x_hbm, len_hbm, out_hbm, len_v, starts_s, *rest):
        bufs = rest[:_NBUF]
        stage, shared, bufb, outb = rest[_NBUF:_NBUF + 4]
        sems = rest[_NBUF + 4:]
        c = lax.axis_index("c")
        s = lax.axis_index("s")
        col0 = c * _HALF

        pltpu.sync_copy(len_hbm, len_v.at[pl.ds(0, 16)])

        total = jnp.int32(0)
        for b in range(_B):
            starts_s[b] = total
            lb = len_v[pl.ds(b, 16)][0]
            total = total + (lb - (lb & (-_CH)))

        lo = lax.shift_right_arithmetic(s * total, 4)
        hi = lax.shift_right_arithmetic((s + 1) * total, 4)

        zero = _zero_vec()

        def seq_body(b, carry):
            start = starts_s[b]
            lb = len_v[pl.ds(b, 16)][0]
            st = lb & (-_CH)
            t_lo = st + jnp.clip(lo - start, 0, lb - st)
            t_hi = st + jnp.clip(hi - start, 0, lb - st)
            nrows = t_hi - t_lo

            for j in range(_NV):
                stage[b, pl.ds(16 * j, 16)] = zero

            @pl.when(nrows > 0)
            def _process():
                a_lo = t_lo & (-8)
                nch = lax.shift_right_arithmetic(
                    t_hi - a_lo + (_R - 1), _RSH)
                ngrp = lax.shift_right_arithmetic(nch + (_NBUF - 1), 2)

                def src(g):
                    t0 = pl.multiple_of(
                        jnp.minimum(a_lo + g * _R, _MAX_LEN - _R), 8)
                    return x_hbm.at[b, pl.ds(t0, _R), pl.ds(col0, _HALF)]

                for i in range(_NBUF):
                    pltpu.async_copy(src(i), bufs[i], sems[i])

                def accum(buf, g, acc):
                    gstart = a_lo + g * _R
                    t0 = jnp.minimum(gstart, _MAX_LEN - _R)
                    k_lo = jnp.maximum(t_lo, gstart) - t0
                    k_hi = jnp.minimum(t_hi, gstart + _R) - t0

                    def row(k, a):
                        return tuple(a[j] + buf[k, pl.ds(16 * j, 16)]
                                     for j in range(_NV))

                    return lax.fori_loop(k_lo, k_hi, row, acc)

                def grp(p, acc):
                    g0 = _NBUF * p
                    for i in range(_NBUF):
                        pltpu.make_async_copy(src(g0 + i), bufs[i],
                                              sems[i]).wait()
                        acc = accum(bufs[i], g0 + i, acc)

                        @pl.when(p + 1 < ngrp)
                        def _issue():
                            pltpu.async_copy(src(g0 + _NBUF + i), bufs[i],
                                             sems[i])
                    return acc

                acc = lax.fori_loop(0, ngrp, grp,
                                    tuple(zero for _ in range(_NV)))
                for j in range(_NV):
                    stage[b, pl.ds(16 * j, 16)] = acc[j]

            return carry

        lax.fori_loop(0, _B, seq_body, jnp.int32(0))

        pltpu.sync_copy(stage, shared.at[s])
        plsc.subcore_barrier()

        @pl.when(s < 2 * _NSLAB)
        def _combine():
            g = s // _NSLAB
            e = s % _NSLAB
            row0 = 8 * g
            cb = 128 * e

            pltpu.sync_copy(
                shared.at[pl.ds(0, _NRANGE), pl.ds(row0, 8), pl.ds(cb, 128)],
                bufb)

            for q in range(8):
                for j in range(8):
                    acc = _zero_vec()
                    for k in range(_NRANGE):
                        acc = acc + bufb[k, q, pl.ds(16 * j, 16)]
                    outb[q, pl.ds(16 * j, 16)] = acc
            pltpu.sync_copy(
                outb,
                out_hbm.at[pl.ds(row0, 8), pl.ds(col0 + cb, 128)])

    return run(x, lengths)


def kernel(input_sequences, sequence_lengths):
    lengths = sequence_lengths.astype(jnp.int32)
    sc = _sc_sums(input_sequences, lengths)
    tc = _tc_sums(input_sequences, lengths)
    return (tc + sc) / lengths.astype(jnp.float32)[:, None]


# speedup vs baseline: 1.4030x; 1.0220x over previous
import functools

import jax
import jax.numpy as jnp
from jax import lax
from jax.experimental import pallas as pl
from jax.experimental.pallas import tpu as pltpu
from jax.experimental.pallas import tpu_sc as plsc

_B = 16
_MAX_LEN = 4096
_D = 1024

_CH = 512
_TBUF = 3
_SHIFT = 256

_HALF = _D // 2
_NV = _HALF // 16
_R = 32
_RSH = 5
_NBUF = 4
_NRANGE = 16
_NSLAB = _HALF // 128


def _zero_vec():
    return jnp.zeros((16,), jnp.float32)


def _tc_sums(x, lengths):
    def body(len_ref, x_ref, o_ref, *rest):
        vbs = rest[:_TBUF]
        sems = rest[_TBUF:]

        pref = []
        tot = jnp.int32(0)
        for b in range(_B):
            pref.append(tot)
            nf = jnp.maximum(
                lax.shift_right_arithmetic(len_ref[b] - _SHIFT, 9), 0)
            tot = tot + nf
        n_chunks = tot

        for b in range(_B):
            o_ref[pl.ds(b, 1), :] = jnp.zeros((1, _D), jnp.float32)

        def locate(c):
            bsel = jnp.int32(0)
            fsel = pref[0]
            for bb in range(1, _B):
                take = c >= pref[bb]
                bsel = jnp.where(take, jnp.int32(bb), bsel)
                fsel = jnp.where(take, pref[bb], fsel)
            return bsel, fsel

        def src(c):
            cc = jnp.clip(c, 0, jnp.maximum(n_chunks - 1, 0))
            bsel, fsel = locate(cc)
            t0 = pl.multiple_of(_CH * (cc - fsel), _CH)
            return x_ref.at[bsel, pl.ds(t0, _CH), :]

        for i in range(_TBUF):
            pltpu.make_async_copy(src(jnp.int32(i)), vbs[i], sems[i]).start()

        ngrp = lax.div(n_chunks + (_TBUF - 1), jnp.int32(_TBUF))

        def grp(p, carry):
            g0 = _TBUF * p
            for i in range(_TBUF):
                c = g0 + i
                pltpu.make_async_copy(src(c), vbs[i], sems[i]).wait()
                s = jnp.sum(vbs[i][...], axis=0, keepdims=True)
                s = jnp.where(c < n_chunks, s, 0.0)
                bsel, _ = locate(jnp.clip(c, 0, jnp.maximum(n_chunks - 1, 0)))
                o_ref[pl.ds(bsel, 1), :] += s

                @pl.when(p + 1 < ngrp)
                def _issue():
                    pltpu.make_async_copy(src(c + _TBUF), vbs[i],
                                          sems[i]).start()
            return carry

        lax.fori_loop(0, ngrp, grp, jnp.int32(0))

    return pl.pallas_call(
        body,
        in_specs=[pl.BlockSpec(memory_space=pltpu.SMEM),
                  pl.BlockSpec(memory_space=pl.ANY)],
        out_specs=pl.BlockSpec(memory_space=pltpu.VMEM),
        out_shape=jax.ShapeDtypeStruct((_B, _D), jnp.float32),
        scratch_shapes=(
            [pltpu.VMEM((_CH, _D), jnp.float32) for _ in range(_TBUF)]
            + [pltpu.SemaphoreType.DMA for _ in range(_TBUF)]
        ),
    )(lengths, x)


def _sc_sums(x, lengths):
    mesh = plsc.VectorSubcoreMesh(core_axis_name="c", subcore_axis_name="s")

    @functools.partial(
        pl.kernel,
        out_type=jax.ShapeDtypeStruct((_B, _D), jnp.float32),
        mesh=mesh,
        scratch_types=(
            [pltpu.VMEM((32,), jnp.int32),
             pltpu.SMEM((16,), jnp.int32)]
            + [pltpu.VMEM((_R, _HALF), jnp.float32) for _ in range(_NBUF)]
            + [pltpu.VMEM((_B, _HALF), jnp.float32),
               pltpu.VMEM_SHARED((_NRANGE, _B, _HALF), jnp.float32),
               pltpu.VMEM((_NRANGE, 8, 128), jnp.float32),
               pltpu.VMEM((8, 128), jnp.float32)]
            + [pltpu.SemaphoreType.DMA for _ in range(_NBUF)]
        ),
    )
    def run(x_hbm, len_hbm, out_hbm, len_v, starts_s, *rest):
        bufs = rest[:_NBUF]
        stage, shared, bufb, outb = rest[_NBUF:_NBUF + 4]
        sems = rest[_NBUF + 4:]
        c = lax.axis_index("c")
        s = lax.axis_index("s")
        col0 = c * _HALF

        pltpu.sync_copy(len_hbm, len_v.at[pl.ds(0, 16)])

        def tc_rows(lb):
            nf = jnp.maximum(lax.shift_right_arithmetic(lb - _SHIFT, 9), 0)
            return lax.shift_left(nf, 9)

        total = jnp.int32(0)
        for b in range(_B):
            starts_s[b] = total
            lb = len_v[pl.ds(b, 16)][0]
            total = total + (lb - tc_rows(lb))

        lo = lax.shift_right_arithmetic(s * total, 4)
        hi = lax.shift_right_arithmetic((s + 1) * total, 4)

        zero = _zero_vec()

        def seq_body(b, carry):
            start = starts_s[b]
            lb = len_v[pl.ds(b, 16)][0]
            st = tc_rows(lb)
            t_lo = st + jnp.clip(lo - start, 0, lb - st)
            t_hi = st + jnp.clip(hi - start, 0, lb - st)
            nrows = t_hi - t_lo

            for j in range(_NV):
                stage[b, pl.ds(16 * j, 16)] = zero

            @pl.when(nrows > 0)
            def _process():
                a_lo = t_lo & (-8)
                nch = lax.shift_right_arithmetic(
                    t_hi - a_lo + (_R - 1), _RSH)
                ngrp = lax.shift_right_arithmetic(nch + (_NBUF - 1), 2)

                def src(g):
                    t0 = pl.multiple_of(
                        jnp.minimum(a_lo + g * _R, _MAX_LEN - _R), 8)
                    return x_hbm.at[b, pl.ds(t0, _R), pl.ds(col0, _HALF)]

                for i in range(_NBUF):
                    pltpu.async_copy(src(i), bufs[i], sems[i])

                def accum(buf, g, acc):
                    gstart = a_lo + g * _R
                    t0 = jnp.minimum(gstart, _MAX_LEN - _R)
                    k_lo = jnp.maximum(t_lo, gstart) - t0
                    k_hi = jnp.minimum(t_hi, gstart + _R) - t0

                    def row(k, a):
                        return tuple(a[j] + buf[k, pl.ds(16 * j, 16)]
                                     for j in range(_NV))

                    return lax.fori_loop(k_lo, k_hi, row, acc)

                def grp(p, acc):
                    g0 = _NBUF * p
                    for i in range(_NBUF):
                        pltpu.make_async_copy(src(g0 + i), bufs[i],
                                              sems[i]).wait()
                        acc = accum(bufs[i], g0 + i, acc)

                        @pl.when(p + 1 < ngrp)
                        def _issue():
                            pltpu.async_copy(src(g0 + _NBUF + i), bufs[i],
                                             sems[i])
                    return acc

                acc = lax.fori_loop(0, ngrp, grp,
                                    tuple(zero for _ in range(_NV)))
                for j in range(_NV):
                    stage[b, pl.ds(16 * j, 16)] = acc[j]

            return carry

        lax.fori_loop(0, _B, seq_body, jnp.int32(0))

        pltpu.sync_copy(stage, shared.at[s])
        plsc.subcore_barrier()

        @pl.when(s < 2 * _NSLAB)
        def _combine():
            g = s // _NSLAB
            e = s % _NSLAB
            row0 = 8 * g
            cb = 128 * e

            pltpu.sync_copy(
                shared.at[pl.ds(0, _NRANGE), pl.ds(row0, 8), pl.ds(cb, 128)],
                bufb)

            for q in range(8):
                for j in range(8):
                    acc = _zero_vec()
                    for k in range(_NRANGE):
                        acc = acc + bufb[k, q, pl.ds(16 * j, 16)]
                    outb[q, pl.ds(16 * j, 16)] = acc
            pltpu.sync_copy(
                outb,
                out_hbm.at[pl.ds(row0, 8), pl.ds(col0 + cb, 128)])

    return run(x, lengths)


def kernel(input_sequences, sequence_lengths):
    lengths = sequence_lengths.astype(jnp.int32)
    sc = _sc_sums(input_sequences, lengths)
    tc = _tc_sums(input_sequences, lengths)
    return (tc + sc) / lengths.astype(jnp.float32)[:, None]


# speedup vs baseline: 1.4600x; 1.0406x over previous
import functools

import jax
import jax.numpy as jnp
from jax import lax
from jax.experimental import pallas as pl
from jax.experimental.pallas import tpu as pltpu
from jax.experimental.pallas import tpu_sc as plsc

_B = 16
_MAX_LEN = 4096
_D = 1024

_CH = 512
_TBUF = 3
_SHIFT = 0

_HALF = _D // 2
_NV = _HALF // 16
_R = 32
_RSH = 5
_NBUF = 4
_NRANGE = 16
_NSLAB = _HALF // 128


def _zero_vec():
    return jnp.zeros((16,), jnp.float32)


def _tc_sums(x, lengths):
    def body(len_ref, x_ref, o_ref, *rest):
        vbs = rest[:_TBUF]
        sems = rest[_TBUF:]

        pref = []
        tot = jnp.int32(0)
        for b in range(_B):
            pref.append(tot)
            nf = jnp.maximum(
                lax.shift_right_arithmetic(len_ref[b] - _SHIFT, 9), 0)
            tot = tot + nf
        n_chunks = tot

        for b in range(_B):
            o_ref[pl.ds(b, 1), :] = jnp.zeros((1, _D), jnp.float32)

        def locate(c):
            bsel = jnp.int32(0)
            fsel = pref[0]
            for bb in range(1, _B):
                take = c >= pref[bb]
                bsel = jnp.where(take, jnp.int32(bb), bsel)
                fsel = jnp.where(take, pref[bb], fsel)
            return bsel, fsel

        def src(c):
            cc = jnp.clip(c, 0, jnp.maximum(n_chunks - 1, 0))
            bsel, fsel = locate(cc)
            t0 = pl.multiple_of(_CH * (cc - fsel), _CH)
            return x_ref.at[bsel, pl.ds(t0, _CH), :]

        for i in range(_TBUF):
            pltpu.make_async_copy(src(jnp.int32(i)), vbs[i], sems[i]).start()

        ngrp = lax.div(n_chunks + (_TBUF - 1), jnp.int32(_TBUF))
        ones = jnp.ones((1, _CH), jnp.float32)

        def grp(p, carry):
            g0 = _TBUF * p
            for i in range(_TBUF):
                c = g0 + i
                pltpu.make_async_copy(src(c), vbs[i], sems[i]).wait()
                s = jnp.dot(ones, vbs[i][...],
                            preferred_element_type=jnp.float32)
                s = jnp.where(c < n_chunks, s, 0.0)
                bsel, _ = locate(jnp.clip(c, 0, jnp.maximum(n_chunks - 1, 0)))
                o_ref[pl.ds(bsel, 1), :] += s

                @pl.when(p + 1 < ngrp)
                def _issue():
                    pltpu.make_async_copy(src(c + _TBUF), vbs[i],
                                          sems[i]).start()
            return carry

        lax.fori_loop(0, ngrp, grp, jnp.int32(0))

    return pl.pallas_call(
        body,
        in_specs=[pl.BlockSpec(memory_space=pltpu.SMEM),
                  pl.BlockSpec(memory_space=pl.ANY)],
        out_specs=pl.BlockSpec(memory_space=pltpu.VMEM),
        out_shape=jax.ShapeDtypeStruct((_B, _D), jnp.float32),
        scratch_shapes=(
            [pltpu.VMEM((_CH, _D), jnp.float32) for _ in range(_TBUF)]
            + [pltpu.SemaphoreType.DMA for _ in range(_TBUF)]
        ),
    )(lengths, x)


def _sc_sums(x, lengths):
    mesh = plsc.VectorSubcoreMesh(core_axis_name="c", subcore_axis_name="s")

    @functools.partial(
        pl.kernel,
        out_type=jax.ShapeDtypeStruct((_B, _D), jnp.float32),
        mesh=mesh,
        scratch_types=(
            [pltpu.VMEM((32,), jnp.int32),
             pltpu.SMEM((16,), jnp.int32)]
            + [pltpu.VMEM((_R, _HALF), jnp.float32) for _ in range(_NBUF)]
            + [pltpu.VMEM((_B, _HALF), jnp.float32),
               pltpu.VMEM_SHARED((_NRANGE, _B, _HALF), jnp.float32),
               pltpu.VMEM((_NRANGE, 8, 128), jnp.float32),
               pltpu.VMEM((8, 128), jnp.float32)]
            + [pltpu.SemaphoreType.DMA for _ in range(_NBUF)]
        ),
    )
    def run(x_hbm, len_hbm, out_hbm, len_v, starts_s, *rest):
        bufs = rest[:_NBUF]
        stage, shared, bufb, outb = rest[_NBUF:_NBUF + 4]
        sems = rest[_NBUF + 4:]
        c = lax.axis_index("c")
        s = lax.axis_index("s")
        col0 = c * _HALF

        pltpu.sync_copy(len_hbm, len_v.at[pl.ds(0, 16)])

        def tc_rows(lb):
            nf = jnp.maximum(lax.shift_right_arithmetic(lb - _SHIFT, 9), 0)
            return lax.shift_left(nf, 9)

        total = jnp.int32(0)
        for b in range(_B):
            starts_s[b] = total
            lb = len_v[pl.ds(b, 16)][0]
            total = total + (lb - tc_rows(lb))

        lo = lax.shift_right_arithmetic(s * total, 4)
        hi = lax.shift_right_arithmetic((s + 1) * total, 4)

        zero = _zero_vec()

        def seq_body(b, carry):
            start = starts_s[b]
            lb = len_v[pl.ds(b, 16)][0]
            st = tc_rows(lb)
            t_lo = st + jnp.clip(lo - start, 0, lb - st)
            t_hi = st + jnp.clip(hi - start, 0, lb - st)
            nrows = t_hi - t_lo

            for j in range(_NV):
                stage[b, pl.ds(16 * j, 16)] = zero

            @pl.when(nrows > 0)
            def _process():
                a_lo = t_lo & (-8)
                nch = lax.shift_right_arithmetic(
                    t_hi - a_lo + (_R - 1), _RSH)
                ngrp = lax.shift_right_arithmetic(nch + (_NBUF - 1), 2)

                def src(g):
                    t0 = pl.multiple_of(
                        jnp.minimum(a_lo + g * _R, _MAX_LEN - _R), 8)
                    return x_hbm.at[b, pl.ds(t0, _R), pl.ds(col0, _HALF)]

                for i in range(_NBUF):
                    pltpu.async_copy(src(i), bufs[i], sems[i])

                def accum(buf, g, acc):
                    gstart = a_lo + g * _R
                    t0 = jnp.minimum(gstart, _MAX_LEN - _R)
                    k_lo = jnp.maximum(t_lo, gstart) - t0
                    k_hi = jnp.minimum(t_hi, gstart + _R) - t0

                    def row(k, a):
                        return tuple(a[j] + buf[k, pl.ds(16 * j, 16)]
                                     for j in range(_NV))

                    return lax.fori_loop(k_lo, k_hi, row, acc)

                def grp(p, acc):
                    g0 = _NBUF * p
                    for i in range(_NBUF):
                        pltpu.make_async_copy(src(g0 + i), bufs[i],
                                              sems[i]).wait()
                        acc = accum(bufs[i], g0 + i, acc)

                        @pl.when(p + 1 < ngrp)
                        def _issue():
                            pltpu.async_copy(src(g0 + _NBUF + i), bufs[i],
                                             sems[i])
                    return acc

                acc = lax.fori_loop(0, ngrp, grp,
                                    tuple(zero for _ in range(_NV)))
                for j in range(_NV):
                    stage[b, pl.ds(16 * j, 16)] = acc[j]

            return carry

        lax.fori_loop(0, _B, seq_body, jnp.int32(0))

        pltpu.sync_copy(stage, shared.at[s])
        plsc.subcore_barrier()

        @pl.when(s < 2 * _NSLAB)
        def _combine():
            g = s // _NSLAB
            e = s % _NSLAB
            row0 = 8 * g
            cb = 128 * e

            pltpu.sync_copy(
                shared.at[pl.ds(0, _NRANGE), pl.ds(row0, 8), pl.ds(cb, 128)],
                bufb)

            for q in range(8):
                for j in range(8):
                    acc = _zero_vec()
                    for k in range(_NRANGE):
                        acc = acc + bufb[k, q, pl.ds(16 * j, 16)]
                    outb[q, pl.ds(16 * j, 16)] = acc
            pltpu.sync_copy(
                outb,
                out_hbm.at[pl.ds(row0, 8), pl.ds(col0 + cb, 128)])

    return run(x, lengths)


def kernel(input_sequences, sequence_lengths):
    lengths = sequence_lengths.astype(jnp.int32)
    sc = _sc_sums(input_sequences, lengths)
    tc = _tc_sums(input_sequences, lengths)
    return (tc + sc) / lengths.astype(jnp.float32)[:, None]
